# baseline (speedup 1.0000x reference)
import functools

import jax
import jax.numpy as jnp
from jax import lax
from jax.experimental import pallas as pl
from jax.experimental.pallas import tpu as pltpu
from jax.experimental.pallas import tpu_sc as plsc

DIM = 64
B = 4096
L = 200
NC = 2
NS = 16
NW = NC * NS
BB = B // NW
NBUF = 4
GROUPS = L // NBUF


def _emb_lookup(table, idx_t):
    mesh = plsc.VectorSubcoreMesh(core_axis_name="c", subcore_axis_name="s")

    @functools.partial(
        pl.kernel,
        mesh=mesh,
        out_type=jax.ShapeDtypeStruct((L, 8, NW, 8, 128), jnp.float32),
        scratch_types=[
            pltpu.VMEM((L, BB), jnp.int32),
            pltpu.VMEM((NBUF, BB, DIM), jnp.float32),
            pltpu.VMEM((NBUF, 8, 8, 128), jnp.float32),
            pltpu.SemaphoreType.DMA((NBUF,)),
            pltpu.SemaphoreType.DMA((NBUF,)),
        ],
        compiler_params=pltpu.CompilerParams(
            use_tc_tiling_on_sc=False, needs_layout_passes=False),
    )
    def k(table_hbm, idxt_hbm, out_hbm, idx_all, rows_v, tbuf, gsem, ssem):
        wid = lax.axis_index("s") * NC + lax.axis_index("c")
        pltpu.sync_copy(idxt_hbm.at[:, pl.ds(wid * BB, BB)], idx_all)

        iota = lax.iota(jnp.int32, 16)
        rowids = [iota + (j0 * 16) for j0 in range(8)]

        def fire(l, b):
            pltpu.async_copy(
                table_hbm.at[idx_all.at[l]], rows_v.at[b], gsem.at[b])

        def wait_g(b):
            pltpu.make_async_copy(
                table_hbm.at[idx_all.at[0]], rows_v.at[b], gsem.at[b]).wait()

        def store(l, b):
            pltpu.async_copy(
                tbuf.at[b], out_hbm.at[l, pl.ds(0, 8), wid], ssem.at[b])

        def wait_s(b):
            pltpu.make_async_copy(
                tbuf.at[b], out_hbm.at[0, pl.ds(0, 8), wid], ssem.at[b]).wait()

        def transpose(b):
            rows2 = rows_v.at[b]

            def tr(d, carry):
                dt = d >> 3
                r = d & 7
                col = jnp.zeros((16,), jnp.int32) + d
                for j0 in range(8):
                    v = plsc.load_gather(rows2, [rowids[j0], col])
                    tbuf[b, dt, r, pl.ds(j0 * 16, 16)] = v
                return carry

            lax.fori_loop(0, 64, tr, 0)

        for b in range(NBUF - 1):
            fire(b, b)

        def group(s, is_first, is_last):
            for bi in range(NBUF):
                l = s * NBUF + bi
                wait_g(bi)
                if not (is_first and bi < NBUF):
                    wait_s(bi)
                transpose(bi)
                store(l, bi)
                bp = (bi - 1) % NBUF
                nxt = l + NBUF - 1
                if is_last and bi > 0:
                    pass
                else:
                    fire(nxt, bp)

        group(0, True, False)

        def body(s, carry):
            group(s, False, False)
            return carry

        lax.fori_loop(1, GROUPS - 1, body, 0)
        group(GROUPS - 1, False, True)

        for b in range(NBUF):
            wait_s(b)

    return k(table, idx_t)


def kernel(inputs, table):
    idx_t = inputs.T.astype(jnp.int32)
    out5 = _emb_lookup(table, idx_t)
    return out5.transpose(2, 4, 0, 1, 3).reshape(B, L, DIM)


# speedup vs baseline: 1.9654x; 1.9654x over previous
import functools

import jax
import jax.numpy as jnp
from jax import lax
from jax.experimental import pallas as pl
from jax.experimental.pallas import tpu as pltpu
from jax.experimental.pallas import tpu_sc as plsc

DIM = 64
PDIM = 128
N = 4096 * 200
NC = 2
NS = 16
NW = NC * NS
PER_W = N // NW
C = 400
CHUNKS = PER_W // C
NBUF = 4
STEPS = CHUNKS // NBUF


def _emb_lookup(table, idx):
    mesh = plsc.VectorSubcoreMesh(core_axis_name="c", subcore_axis_name="s")

    @functools.partial(
        pl.kernel,
        mesh=mesh,
        out_type=jax.ShapeDtypeStruct((N, PDIM), jnp.float32),
        scratch_types=[
            pltpu.VMEM((PER_W,), jnp.int32),
            pltpu.VMEM((NBUF, C, DIM), jnp.float32),
            pltpu.SemaphoreType.DMA((NBUF,)),
            pltpu.SemaphoreType.DMA((NBUF,)),
        ],
        compiler_params=pltpu.CompilerParams(use_tc_tiling_on_sc=False),
    )
    def k(table_hbm, idx_hbm, out_hbm, idx_all, rows_v, gsem, ssem):
        wid = lax.axis_index("s") * NC + lax.axis_index("c")
        base = wid * PER_W
        pltpu.sync_copy(idx_hbm.at[pl.ds(base, PER_W)], idx_all)

        def fire(g, b):
            pltpu.async_copy(
                table_hbm.at[idx_all.at[pl.ds(g * C, C)]],
                rows_v.at[b], gsem.at[b])

        def wait_g(b):
            pltpu.make_async_copy(
                table_hbm.at[idx_all.at[pl.ds(0, C)]],
                rows_v.at[b], gsem.at[b]).wait()

        def store(g, b):
            pltpu.async_copy(
                rows_v.at[b],
                out_hbm.at[pl.ds(base + g * C, C), pl.ds(0, DIM)],
                ssem.at[b])

        def wait_s(b):
            pltpu.make_async_copy(
                rows_v.at[b],
                out_hbm.at[pl.ds(0, C), pl.ds(0, DIM)],
                ssem.at[b]).wait()

        for b in range(NBUF - 1):
            fire(b, b)

        def group(s, is_first, is_last):
            for bi in range(NBUF):
                g = s * NBUF + bi
                wait_g(bi)
                store(g, bi)
                bp = (bi - 1) % NBUF
                nxt = g + NBUF - 1
                if is_first and bi == 0:
                    fire(nxt, bp)
                elif is_last and bi > 0:
                    pass
                else:
                    wait_s(bp)
                    fire(nxt, bp)

        group(0, True, False)

        def body(s, carry):
            group(s, False, False)
            return carry

        lax.fori_loop(1, STEPS - 1, body, 0)
        group(STEPS - 1, False, True)

        for b in range(NBUF):
            wait_s(b)

    return k(table, idx)


def kernel(inputs, table):
    idx = inputs.reshape(-1).astype(jnp.int32)
    out_p = _emb_lookup(table, idx)
    return out_p[:, :DIM].reshape(inputs.shape + (DIM,))
